# Initial kernel scaffold; baseline (speedup 1.0000x reference)
import functools

import jax
import jax.numpy as jnp
from jax import lax
from jax.experimental import pallas as pl
from jax.experimental.pallas import tpu as pltpu
from jax.experimental.pallas import tpu_sc as plsc

N = 10000
D = 128
E = 320000
NC = 2
NS = 16
NW = NC * NS
CHUNK = 128
NCH = -(-E // (NW * CHUNK))
EP = NW * CHUNK * NCH
NDEG = 10240
RSTRIPE = N // NS
RB = 1000
GRID = N // RB


def _mesh():
    return plsc.VectorSubcoreMesh(core_axis_name="c", subcore_axis_name="s")


@functools.partial(
    pl.kernel,
    out_type=(jax.ShapeDtypeStruct((NDEG,), jnp.float32),
              jax.ShapeDtypeStruct((NDEG,), jnp.float32)),
    mesh=_mesh(),
    scratch_types=[
        pltpu.VMEM((CHUNK,), jnp.int32),
        pltpu.VMEM((CHUNK,), jnp.float32),
        pltpu.VMEM((CHUNK,), jnp.float32),
        pltpu.VMEM_SHARED((NDEG,), jnp.float32),
    ],
)
def _deg_kernel(dst_hbm, w_hbm, out0, out1, didx, wv, zb, deg_sh):
    c = lax.axis_index("c")
    s = lax.axis_index("s")
    wid = c * NS + s

    for j in range(8):
        zb[pl.ds(j * 16, 16)] = jnp.zeros((16,), jnp.float32)
    for k in range(5):
        pltpu.sync_copy(zb, deg_sh.at[pl.ds(s * 640 + k * CHUNK, CHUNK)])
    plsc.subcore_barrier()

    def body(g, carry):
        base = (wid * NCH + g) * CHUNK
        pltpu.sync_copy(dst_hbm.at[pl.ds(base, CHUNK)], didx)
        pltpu.sync_copy(w_hbm.at[pl.ds(base, CHUNK)], wv)
        pltpu.sync_copy(wv, deg_sh.at[didx], add=True)
        return carry

    lax.fori_loop(0, NCH, body, 0)
    plsc.subcore_barrier()

    @pl.when(c == 0)
    def _():
        pltpu.sync_copy(deg_sh.at[pl.ds(s * 640, 640)], out0.at[pl.ds(s * 640, 640)])

    @pl.when(c == 1)
    def _():
        pltpu.sync_copy(deg_sh.at[pl.ds(s * 640, 640)], out1.at[pl.ds(s * 640, 640)])


@functools.partial(
    pl.kernel,
    out_type=(jax.ShapeDtypeStruct((N, D), jnp.float32),
              jax.ShapeDtypeStruct((N, D), jnp.float32)),
    mesh=_mesh(),
    scratch_types=[
        pltpu.VMEM((CHUNK, D), jnp.float32),
        pltpu.VMEM((CHUNK,), jnp.int32),
        pltpu.VMEM((CHUNK,), jnp.int32),
        pltpu.VMEM((CHUNK,), jnp.float32),
        pltpu.VMEM_SHARED((N, D), jnp.float32),
        pltpu.SemaphoreType.DMA,
    ],
)
def _scatter_kernel(hp_hbm, src_hbm, dst_hbm, w_hbm, out0, out1,
                    rows, sidx, didx, wv, s_sh, sem):
    c = lax.axis_index("c")
    s = lax.axis_index("s")
    wid = c * NS + s

    def zrow(r, carry):
        for j in range(8):
            rows[r, pl.ds(j * 16, 16)] = jnp.zeros((16,), jnp.float32)
        return carry

    lax.fori_loop(0, CHUNK, zrow, 0)
    base = s * RSTRIPE
    for k in range(4):
        pltpu.sync_copy(rows, s_sh.at[pl.ds(base + k * CHUNK, CHUNK)])
    pltpu.sync_copy(rows.at[pl.ds(0, RSTRIPE - 4 * CHUNK)],
                    s_sh.at[pl.ds(base + 4 * CHUNK, RSTRIPE - 4 * CHUNK)])
    plsc.subcore_barrier()

    def body(g, carry):
        eb = (wid * NCH + g) * CHUNK
        pltpu.sync_copy(src_hbm.at[pl.ds(eb, CHUNK)], sidx)
        pltpu.sync_copy(dst_hbm.at[pl.ds(eb, CHUNK)], didx)
        pltpu.sync_copy(w_hbm.at[pl.ds(eb, CHUNK)], wv)
        pltpu.async_copy(hp_hbm.at[sidx], rows, sem).wait()

        def srow(r, cc):
            ws = wv[r]
            for j in range(8):
                sl = pl.ds(j * 16, 16)
                rows[r, sl] = rows[r, sl] * ws
            return cc

        lax.fori_loop(0, CHUNK, srow, 0)
        pltpu.sync_copy(rows, s_sh.at[didx], add=True)
        return carry

    lax.fori_loop(0, NCH, body, 0)
    plsc.subcore_barrier()

    @pl.when(c == 0)
    def _():
        pltpu.sync_copy(s_sh.at[pl.ds(s * RSTRIPE, RSTRIPE)],
                        out0.at[pl.ds(s * RSTRIPE, RSTRIPE)])

    @pl.when(c == 1)
    def _():
        pltpu.sync_copy(s_sh.at[pl.ds(s * RSTRIPE, RSTRIPE)],
                        out1.at[pl.ds(s * RSTRIPE, RSTRIPE)])


def _tc1_body(d1, d2, x_ref, w_ref, dinv_ref, h1p_ref):
    deg = d1[...] + d2[...] + 1.0
    dinv = jnp.where(deg > 0, lax.rsqrt(jnp.maximum(deg, 1e-12)), 0.0)
    dinv_ref[...] = dinv
    h = jnp.dot(x_ref[...], w_ref[...], preferred_element_type=jnp.float32)
    h1p_ref[...] = h * dinv


def _tc2_body(dinv_ref, h1p_ref, sa, sb, b1_ref, w2_ref, h2p_ref):
    dinv = dinv_ref[...]
    h1 = jnp.maximum(dinv * (sa[...] + sb[...] + h1p_ref[...]) + b1_ref[...], 0.0)
    h2p_ref[...] = jnp.dot(h1, w2_ref[...], preferred_element_type=jnp.float32) * dinv


def _tc3_body(dinv_ref, h2p_ref, sa, sb, b2_ref, wm_ref, bm_ref,
              out_ref, acc):
    i = pl.program_id(0)
    dinv = dinv_ref[...]
    h2 = jnp.maximum(dinv * (sa[...] + sb[...] + h2p_ref[...]) + b2_ref[...], 0.0)
    psum = jnp.sum(h2, axis=0, keepdims=True)

    @pl.when(i == 0)
    def _():
        acc[...] = psum

    @pl.when(i > 0)
    def _():
        acc[...] = acc[...] + psum

    @pl.when(i == GRID - 1)
    def _():
        out_ref[...] = (jnp.dot(acc[...] * (1.0 / N), wm_ref[...],
                                preferred_element_type=jnp.float32)
                        + bm_ref[...])


def _row_spec(width):
    return pl.BlockSpec((RB, width), lambda i: (i, 0))


def _full_spec(shape):
    return pl.BlockSpec(shape, lambda i: (0, 0))


_tc1 = pl.pallas_call(
    _tc1_body,
    grid=(GRID,),
    in_specs=[_row_spec(1), _row_spec(1), _row_spec(D), _full_spec((D, D))],
    out_specs=[_row_spec(1), _row_spec(D)],
    out_shape=[jax.ShapeDtypeStruct((N, 1), jnp.float32),
               jax.ShapeDtypeStruct((N, D), jnp.float32)],
)

_tc2 = pl.pallas_call(
    _tc2_body,
    grid=(GRID,),
    in_specs=[_row_spec(1), _row_spec(D), _row_spec(D), _row_spec(D),
              _full_spec((1, D)), _full_spec((D, D))],
    out_specs=pl.BlockSpec((RB, D), lambda i: (i, 0)),
    out_shape=jax.ShapeDtypeStruct((N, D), jnp.float32),
)


def _make_tc3(nclass):
    return pl.pallas_call(
        _tc3_body,
        grid=(GRID,),
        in_specs=[_row_spec(1), _row_spec(D), _row_spec(D), _row_spec(D),
                  _full_spec((1, D)), _full_spec((D, nclass)),
                  _full_spec((1, nclass))],
        out_specs=pl.BlockSpec((1, nclass), lambda i: (0, 0)),
        out_shape=jax.ShapeDtypeStruct((1, nclass), jnp.float32),
        scratch_shapes=[pltpu.VMEM((1, D), jnp.float32)],
    )


def kernel(x, edge_index, edge_attr, W1, b1, W2, b2, Wm, bm):
    src = edge_index[0].astype(jnp.int32)
    dst = edge_index[1].astype(jnp.int32)
    w = edge_attr.astype(jnp.float32)
    pad = EP - E
    srcp = jnp.concatenate([src, jnp.zeros((pad,), jnp.int32)])
    dstp = jnp.concatenate([dst, jnp.zeros((pad,), jnp.int32)])
    wp = jnp.concatenate([w, jnp.zeros((pad,), jnp.float32)])

    dega, degb = _deg_kernel(dstp, wp)
    deg1 = dega[:N].reshape(N, 1)
    deg2 = degb[:N].reshape(N, 1)

    dinv, h1p = _tc1(deg1, deg2, x, W1)
    s1a, s1b = _scatter_kernel(h1p, srcp, dstp, wp)
    h2p = _tc2(dinv, h1p, s1a, s1b, b1.reshape(1, D), W2)
    s2a, s2b = _scatter_kernel(h2p, srcp, dstp, wp)
    nclass = Wm.shape[1]
    out = _make_tc3(nclass)(dinv, h2p, s2a, s2b, b2.reshape(1, D),
                            Wm, bm.reshape(1, nclass))
    return out.reshape(nclass)


# speedup vs baseline: 8.8326x; 8.8326x over previous
import functools

import jax
import jax.numpy as jnp
from jax import lax
from jax.experimental import pallas as pl
from jax.experimental.pallas import tpu as pltpu
from jax.experimental.pallas import tpu_sc as plsc

N = 10000
D = 128
E = 320000
NC = 2
NS = 16
NW = NC * NS
CHUNK = 128
NCH = -(-E // (NW * CHUNK))
EP = NW * CHUNK * NCH
NDEG = 10240
NPAD = 10240
RSTRIPE = NPAD // NS
RB = 1000
GRID = N // RB


def _mesh():
    return plsc.VectorSubcoreMesh(core_axis_name="c", subcore_axis_name="s",
                                  num_cores=NC, num_subcores=NS)


@functools.partial(
    pl.kernel,
    out_type=(jax.ShapeDtypeStruct((NDEG,), jnp.float32),
              jax.ShapeDtypeStruct((NDEG,), jnp.float32)),
    mesh=_mesh(),
    scratch_types=[
        pltpu.VMEM((CHUNK,), jnp.int32),
        pltpu.VMEM((CHUNK,), jnp.float32),
        pltpu.VMEM((CHUNK,), jnp.float32),
        pltpu.VMEM_SHARED((NDEG,), jnp.float32),
    ],
)
def _deg_kernel(dst_hbm, w_hbm, out0, out1, didx, wv, zb, deg_sh):
    c = lax.axis_index("c")
    s = lax.axis_index("s")
    wid = c * NS + s

    for j in range(8):
        zb[pl.ds(j * 16, 16)] = jnp.zeros((16,), jnp.float32)
    for k in range(5):
        pltpu.sync_copy(zb, deg_sh.at[pl.ds(s * 640 + k * CHUNK, CHUNK)])
    plsc.subcore_barrier()

    def body(g, carry):
        base = (wid * NCH + g) * CHUNK
        pltpu.sync_copy(dst_hbm.at[pl.ds(base, CHUNK)], didx)
        pltpu.sync_copy(w_hbm.at[pl.ds(base, CHUNK)], wv)
        pltpu.sync_copy(wv, deg_sh.at[didx], add=True)
        return carry

    lax.fori_loop(0, NCH, body, 0)
    plsc.subcore_barrier()

    @pl.when(c == 0)
    def _():
        pltpu.sync_copy(deg_sh.at[pl.ds(s * 640, 640)], out0.at[pl.ds(s * 640, 640)])

    @pl.when(c == 1)
    def _():
        pltpu.sync_copy(deg_sh.at[pl.ds(s * 640, 640)], out1.at[pl.ds(s * 640, 640)])


@functools.partial(
    pl.kernel,
    out_type=(jax.ShapeDtypeStruct((NPAD, D), jnp.float32),
              jax.ShapeDtypeStruct((NPAD, D), jnp.float32)),
    mesh=_mesh(),
    scratch_types=[
        pltpu.VMEM((CHUNK, D), jnp.float32),
        pltpu.VMEM((CHUNK,), jnp.int32),
        pltpu.VMEM((CHUNK,), jnp.int32),
        pltpu.VMEM((CHUNK,), jnp.float32),
        pltpu.VMEM_SHARED((NPAD, D), jnp.float32),
        pltpu.SemaphoreType.DMA,
    ],
)
def _scatter_kernel(hp_hbm, src_hbm, dst_hbm, w_hbm, out0, out1,
                    rows, sidx, didx, wv, s_sh, sem):
    c = lax.axis_index("c")
    s = lax.axis_index("s")
    wid = c * NS + s

    def zrow(r, carry):
        for j in range(8):
            rows[r, pl.ds(j * 16, 16)] = jnp.zeros((16,), jnp.float32)
        return carry

    lax.fori_loop(0, CHUNK, zrow, 0)
    base = s * RSTRIPE
    for k in range(RSTRIPE // CHUNK):
        pltpu.sync_copy(rows, s_sh.at[pl.ds(base + k * CHUNK, CHUNK)])
    plsc.subcore_barrier()

    def body(g, carry):
        eb = (wid * NCH + g) * CHUNK
        pltpu.sync_copy(src_hbm.at[pl.ds(eb, CHUNK)], sidx)
        pltpu.sync_copy(dst_hbm.at[pl.ds(eb, CHUNK)], didx)
        pltpu.sync_copy(w_hbm.at[pl.ds(eb, CHUNK)], wv)
        pltpu.async_copy(hp_hbm.at[sidx], rows, sem).wait()

        def sgrp(t, cc):
            wvec = wv[pl.ds(t * 16, 16)]
            for l in range(16):
                ws = wvec[l]
                r = t * 16 + l
                for j in range(8):
                    sl = pl.ds(j * 16, 16)
                    rows[r, sl] = rows[r, sl] * ws
            return cc

        lax.fori_loop(0, CHUNK // 16, sgrp, 0)
        pltpu.sync_copy(rows, s_sh.at[didx], add=True)
        return carry

    lax.fori_loop(0, NCH, body, 0)
    plsc.subcore_barrier()

    @pl.when(c == 0)
    def _():
        pltpu.sync_copy(s_sh.at[pl.ds(s * RSTRIPE, RSTRIPE)],
                        out0.at[pl.ds(s * RSTRIPE, RSTRIPE)])

    @pl.when(c == 1)
    def _():
        pltpu.sync_copy(s_sh.at[pl.ds(s * RSTRIPE, RSTRIPE)],
                        out1.at[pl.ds(s * RSTRIPE, RSTRIPE)])


def _tc1_body(d1, d2, x_ref, w_ref, dinv_ref, h1p_ref):
    deg = d1[...] + d2[...] + 1.0
    dinv = jnp.where(deg > 0, lax.rsqrt(jnp.maximum(deg, 1e-12)), 0.0)
    dinv_ref[...] = dinv
    h = jnp.dot(x_ref[...], w_ref[...], preferred_element_type=jnp.float32)
    h1p_ref[...] = h * dinv


def _tc2_body(dinv_ref, h1p_ref, sa, sb, b1_ref, w2_ref, h2p_ref):
    dinv = dinv_ref[...]
    h1 = jnp.maximum(dinv * (sa[...] + sb[...] + h1p_ref[...]) + b1_ref[...], 0.0)
    h2p_ref[...] = jnp.dot(h1, w2_ref[...], preferred_element_type=jnp.float32) * dinv


def _tc3_body(dinv_ref, h2p_ref, sa, sb, b2_ref, wm_ref, bm_ref,
              out_ref, acc):
    i = pl.program_id(0)
    dinv = dinv_ref[...]
    h2 = jnp.maximum(dinv * (sa[...] + sb[...] + h2p_ref[...]) + b2_ref[...], 0.0)
    psum = jnp.sum(h2, axis=0, keepdims=True)

    @pl.when(i == 0)
    def _():
        acc[...] = psum

    @pl.when(i > 0)
    def _():
        acc[...] = acc[...] + psum

    @pl.when(i == GRID - 1)
    def _():
        out_ref[...] = (jnp.dot(acc[...] * (1.0 / N), wm_ref[...],
                                preferred_element_type=jnp.float32)
                        + bm_ref[...])


def _row_spec(width):
    return pl.BlockSpec((RB, width), lambda i: (i, 0))


def _full_spec(shape):
    return pl.BlockSpec(shape, lambda i: (0, 0))


_tc1 = pl.pallas_call(
    _tc1_body,
    grid=(GRID,),
    in_specs=[_row_spec(1), _row_spec(1), _row_spec(D), _full_spec((D, D))],
    out_specs=[_row_spec(1), _row_spec(D)],
    out_shape=[jax.ShapeDtypeStruct((N, 1), jnp.float32),
               jax.ShapeDtypeStruct((N, D), jnp.float32)],
)

_tc2 = pl.pallas_call(
    _tc2_body,
    grid=(GRID,),
    in_specs=[_row_spec(1), _row_spec(D), _row_spec(D), _row_spec(D),
              _full_spec((1, D)), _full_spec((D, D))],
    out_specs=pl.BlockSpec((RB, D), lambda i: (i, 0)),
    out_shape=jax.ShapeDtypeStruct((N, D), jnp.float32),
)


def _make_tc3(nclass):
    return pl.pallas_call(
        _tc3_body,
        grid=(GRID,),
        in_specs=[_row_spec(1), _row_spec(D), _row_spec(D), _row_spec(D),
                  _full_spec((1, D)), _full_spec((D, nclass)),
                  _full_spec((1, nclass))],
        out_specs=pl.BlockSpec((1, nclass), lambda i: (0, 0)),
        out_shape=jax.ShapeDtypeStruct((1, nclass), jnp.float32),
        scratch_shapes=[pltpu.VMEM((1, D), jnp.float32)],
    )


def kernel(x, edge_index, edge_attr, W1, b1, W2, b2, Wm, bm):
    src = edge_index[0].astype(jnp.int32)
    dst = edge_index[1].astype(jnp.int32)
    w = edge_attr.astype(jnp.float32)
    pad = EP - E
    srcp = jnp.concatenate([src, jnp.zeros((pad,), jnp.int32)])
    dstp = jnp.concatenate([dst, jnp.zeros((pad,), jnp.int32)])
    wp = jnp.concatenate([w, jnp.zeros((pad,), jnp.float32)])

    dega, degb = _deg_kernel(dstp, wp)
    deg1 = dega[:N].reshape(N, 1)
    deg2 = degb[:N].reshape(N, 1)

    dinv, h1p = _tc1(deg1, deg2, x, W1)
    s1a, s1b = _scatter_kernel(h1p, srcp, dstp, wp)
    h2p = _tc2(dinv, h1p, s1a[:N], s1b[:N], b1.reshape(1, D), W2)
    s2a, s2b = _scatter_kernel(h2p, srcp, dstp, wp)
    nclass = Wm.shape[1]
    out = _make_tc3(nclass)(dinv, h2p, s2a[:N], s2b[:N], b2.reshape(1, D),
                            Wm, bm.reshape(1, nclass))
    return out.reshape(nclass)
